# baseline (speedup 1.0000x reference)
import functools

import jax
import jax.numpy as jnp
from jax import lax
from jax.experimental import pallas as pl
from jax.experimental.pallas import tpu as pltpu
from jax.experimental.pallas import tpu_sc as plsc

NUM_USERS = 3000
NUM_ITEMS = 7000
N_TOTAL = NUM_USERS + NUM_ITEMS
D = 128
LAYERS = 3
NNZ = 320000

NC = 2
NS = 16
L = 16
NB = 4

N_PAD = 10240
R_CORE = N_PAD // NC
R_TILE = R_CORE // NS
DUMMY = R_CORE
ACC_R = R_CORE + 8
CH = 128
E_PAD = NNZ + (NB + 1) * CH


def _layer_body(ego, edat, vals, prm, out,
                ebuf0, ebuf1, ebuf2, ebuf3, vbuf0, vbuf1, vbuf2, vbuf3,
                lidx0, lidx1, lidx2, lidx3, gbuf0, gbuf1, gbuf2, gbuf3,
                zblk, prmv, acc,
                se0, se1, se2, se3, sg0, sg1, sg2, sg3,
                ss0, ss1, ss2, ss3):
    ebuf = (ebuf0, ebuf1, ebuf2, ebuf3)
    vbuf = (vbuf0, vbuf1, vbuf2, vbuf3)
    lidxv = (lidx0, lidx1, lidx2, lidx3)
    gbuf = (gbuf0, gbuf1, gbuf2, gbuf3)
    sem_e = (se0, se1, se2, se3)
    sem_g = (sg0, sg1, sg2, sg3)
    sem_s = (ss0, ss1, ss2, ss3)

    c = lax.axis_index("c")
    s = lax.axis_index("s")

    z16 = jnp.zeros((L,), jnp.float32)
    for r in range(L):
        for g in range(D // L):
            zblk[r, pl.ds(g * L, L)] = z16
    for k in range(R_TILE // L):
        pltpu.async_copy(zblk, acc.at[pl.ds(s * R_TILE + k * L, L)], se0)

    @pl.when(s == NS - 1)
    def _zero_dummy():
        pltpu.async_copy(zblk, acc.at[pl.ds(ACC_R - L, L)], se0)

    pltpu.sync_copy(prm, prmv)
    split = prmv[...][0]
    for k in range(R_TILE // L):
        pltpu.make_async_copy(zblk, acc.at[pl.ds(s * R_TILE + k * L, L)],
                              se0).wait()

    @pl.when(s == NS - 1)
    def _drain_dummy():
        pltpu.make_async_copy(zblk, acc.at[pl.ds(ACC_R - L, L)], se0).wait()

    plsc.subcore_barrier()

    estart = jnp.where(c == 0, 0, split)
    eend = jnp.where(c == 0, split, NNZ)
    astart = (estart // CH) * CH
    nch = (eend - astart + CH - 1) // CH
    nch_per = (nch + NS - 1) // NS
    k0 = s * nch_per
    nt = jnp.maximum(jnp.minimum(k0 + nch_per, nch) - k0, 0)
    rowbase = c * R_CORE
    lane = lax.iota(jnp.int32, L)

    def chunk_base(t):
        return pl.multiple_of(astart + (k0 + t) * CH, CH)

    def drain_scatter(p):
        pltpu.make_async_copy(gbuf[p], acc.at[lidxv[p]], sem_s[p]).wait()

    def stage_a(t, p):
        @pl.when(t < nt)
        def _():
            base = chunk_base(t)
            pltpu.async_copy(edat.at[:, pl.ds(base, CH)], ebuf[p], sem_e[p])
            pltpu.async_copy(vals.at[pl.ds(base, CH)], vbuf[p], sem_e[p])

    def stage_b(t, p, first=False):
        @pl.when(t < nt)
        def _():
            base = chunk_base(t)
            if not first:
                @pl.when(t >= NB)
                def _drain():
                    drain_scatter(p)
            pltpu.make_async_copy(edat.at[:, pl.ds(base, CH)],
                                  ebuf[p], sem_e[p]).wait()
            pltpu.make_async_copy(vals.at[pl.ds(base, CH)],
                                  vbuf[p], sem_e[p]).wait()
            for j in range(CH // L):
                sl = pl.ds(j * L, L)
                pos = base + j * L + lane
                r16 = ebuf[p][1, sl]
                m = (pos >= estart) & (pos < eend)
                lidxv[p][sl] = jnp.where(m, r16 - rowbase, DUMMY)
            pltpu.async_copy(ego.at[ebuf[p].at[0]], gbuf[p], sem_g[p])

    def stage_c(t, p):
        @pl.when(t < nt)
        def _():
            pltpu.make_async_copy(ego.at[ebuf[p].at[0]], gbuf[p],
                                  sem_g[p]).wait()
            for j in range(CH // L):
                vv = vbuf[p][pl.ds(j * L, L)]
                for i in range(L):
                    sv = vv[i]
                    e = j * L + i
                    for g in range(D // L):
                        sl = pl.ds(g * L, L)
                        gbuf[p][e, sl] = gbuf[p][e, sl] * sv
            pltpu.async_copy(gbuf[p], acc.at[lidxv[p]], sem_s[p], add=True)

    for t in range(NB):
        stage_a(t, t)
    stage_b(0, 0, first=True)
    stage_b(1, 1, first=True)

    def body3(u, carry):
        for q in range(NB):
            t = NB * u + q
            stage_b(t + 2, (q + 2) % NB)
            stage_c(t, q)
            stage_a(t + NB, q)
        return carry

    lax.fori_loop(0, (nt + NB - 1) // NB, body3, 0)

    t_last = nt - 1
    for p in range(NB):
        tp = t_last - jnp.mod(t_last - p, NB)

        @pl.when(tp >= 0)
        def _drain_tail(p=p):
            drain_scatter(p)

    plsc.subcore_barrier()
    pltpu.sync_copy(acc.at[pl.ds(s * R_TILE, R_TILE)],
                    out.at[pl.ds(c * R_CORE + s * R_TILE, R_TILE)])


@jax.jit
def _layer(ego, edat, vals, prm):
    mesh = plsc.VectorSubcoreMesh(core_axis_name="c", subcore_axis_name="s")
    return pl.kernel(
        _layer_body,
        out_type=jax.ShapeDtypeStruct((N_PAD, D), jnp.float32),
        mesh=mesh,
        scratch_types=(
            [pltpu.VMEM((2, CH), jnp.int32)] * NB
            + [pltpu.VMEM((CH,), jnp.float32)] * NB
            + [pltpu.VMEM((CH,), jnp.int32)] * NB
            + [pltpu.VMEM((CH, D), jnp.float32)] * NB
            + [pltpu.VMEM((L, D), jnp.float32),
               pltpu.VMEM((L,), jnp.int32),
               pltpu.VMEM_SHARED((ACC_R, D), jnp.float32)]
            + [pltpu.SemaphoreType.DMA] * (3 * NB)
        ),
    )(ego, edat, vals, prm)


def _mean_body(a, b, c, o):
    o[...] = (a[...] + b[...] + c[...]) * (1.0 / 3.0)


@jax.jit
def _mean3(e1, e2, e3):
    blk = pl.BlockSpec((1024, D), lambda i: (i, 0))
    return pl.pallas_call(
        _mean_body,
        grid=(N_PAD // 1024,),
        in_specs=[blk, blk, blk],
        out_specs=blk,
        out_shape=jax.ShapeDtypeStruct((N_PAD, D), jnp.float32),
    )(e1, e2, e3)


def kernel(user_emb, item_emb, adj_rows, adj_cols, adj_vals):
    ego0 = jnp.concatenate(
        [user_emb, item_emb,
         jnp.zeros((N_PAD - N_TOTAL, D), jnp.float32)], axis=0)
    rows = adj_rows.astype(jnp.int32)
    cols = adj_cols.astype(jnp.int32)
    vals = adj_vals.astype(jnp.float32)
    pad = E_PAD - NNZ
    rows_p = jnp.concatenate([rows, jnp.full((pad,), N_TOTAL - 1, jnp.int32)])
    cols_p = jnp.concatenate([cols, jnp.zeros((pad,), jnp.int32)])
    vals_p = jnp.concatenate([vals, jnp.zeros((pad,), jnp.float32)])
    edat = jnp.stack([cols_p, rows_p], axis=0)
    split = jnp.searchsorted(rows, R_CORE, side="left").astype(jnp.int32)
    prm = jnp.full((L,), split, jnp.int32)

    ego = ego0
    outs = []
    for _ in range(LAYERS):
        ego = _layer(ego, edat, vals_p, prm)
        outs.append(ego)
    mean = _mean3(*outs)
    return (mean[:NUM_USERS], mean[NUM_USERS:N_TOTAL])


# speedup vs baseline: 1.0553x; 1.0553x over previous
import functools

import jax
import jax.numpy as jnp
from jax import lax
from jax.experimental import pallas as pl
from jax.experimental.pallas import tpu as pltpu
from jax.experimental.pallas import tpu_sc as plsc

NUM_USERS = 3000
NUM_ITEMS = 7000
N_TOTAL = NUM_USERS + NUM_ITEMS
D = 128
LAYERS = 3
NNZ = 320000

NC = 2
NS = 16
L = 16
NB = 3

N_PAD = 10240
R_CORE = N_PAD // NC
R_TILE = R_CORE // NS
DUMMY = R_CORE
ACC_R = R_CORE + 8
CH = 128
E_PAD = NNZ + (NB + 1) * CH


def _layer_body(ego, edat, vals, prm, out,
                ebuf0, ebuf1, ebuf2, vbuf0, vbuf1, vbuf2,
                lidx0, lidx1, lidx2, gbuf0, gbuf1, gbuf2,
                zblk, prmv, acc,
                se0, se1, se2, sg0, sg1, sg2, ss0, ss1, ss2):
    ebuf = (ebuf0, ebuf1, ebuf2)
    vbuf = (vbuf0, vbuf1, vbuf2)
    lidxv = (lidx0, lidx1, lidx2)
    gbuf = (gbuf0, gbuf1, gbuf2)
    sem_e = (se0, se1, se2)
    sem_g = (sg0, sg1, sg2)
    sem_s = (ss0, ss1, ss2)

    c = lax.axis_index("c")
    s = lax.axis_index("s")

    z16 = jnp.zeros((L,), jnp.float32)
    for r in range(L):
        for g in range(D // L):
            zblk[r, pl.ds(g * L, L)] = z16
    for k in range(R_TILE // L):
        pltpu.async_copy(zblk, acc.at[pl.ds(s * R_TILE + k * L, L)], se0)

    @pl.when(s == NS - 1)
    def _zero_dummy():
        pltpu.async_copy(zblk, acc.at[pl.ds(ACC_R - L, L)], se0)

    pltpu.sync_copy(prm, prmv)
    split = prmv[...][0]
    for k in range(R_TILE // L):
        pltpu.make_async_copy(zblk, acc.at[pl.ds(s * R_TILE + k * L, L)],
                              se0).wait()

    @pl.when(s == NS - 1)
    def _drain_dummy():
        pltpu.make_async_copy(zblk, acc.at[pl.ds(ACC_R - L, L)], se0).wait()

    plsc.subcore_barrier()

    estart = jnp.where(c == 0, 0, split)
    eend = jnp.where(c == 0, split, NNZ)
    astart = (estart // CH) * CH
    nch = (eend - astart + CH - 1) // CH
    nch_per = (nch + NS - 1) // NS
    k0 = s * nch_per
    nt = jnp.maximum(jnp.minimum(k0 + nch_per, nch) - k0, 0)
    rowbase = c * R_CORE
    lane = lax.iota(jnp.int32, L)

    def chunk_base(t):
        return pl.multiple_of(astart + (k0 + t) * CH, CH)

    def drain_scatter(p):
        pltpu.make_async_copy(gbuf[p], acc.at[lidxv[p]], sem_s[p]).wait()

    def stage_a(t, p):
        @pl.when(t < nt)
        def _():
            base = chunk_base(t)
            pltpu.async_copy(edat.at[:, pl.ds(base, CH)], ebuf[p], sem_e[p])
            pltpu.async_copy(vals.at[pl.ds(base, CH)], vbuf[p], sem_e[p])

    def stage_b(t, p, first=False):
        @pl.when(t < nt)
        def _():
            base = chunk_base(t)
            if not first:
                @pl.when(t >= NB)
                def _drain():
                    drain_scatter(p)
            pltpu.make_async_copy(edat.at[:, pl.ds(base, CH)],
                                  ebuf[p], sem_e[p]).wait()
            pltpu.make_async_copy(vals.at[pl.ds(base, CH)],
                                  vbuf[p], sem_e[p]).wait()
            for j in range(CH // L):
                sl = pl.ds(j * L, L)
                pos = base + j * L + lane
                r16 = ebuf[p][1, sl]
                m = (pos >= estart) & (pos < eend)
                lidxv[p][sl] = jnp.where(m, r16 - rowbase, DUMMY)
            pltpu.async_copy(ego.at[ebuf[p].at[0]], gbuf[p], sem_g[p])

    def stage_c(t, p):
        @pl.when(t < nt)
        def _():
            pltpu.make_async_copy(ego.at[ebuf[p].at[0]], gbuf[p],
                                  sem_g[p]).wait()
            for j in range(CH // L):
                vv = vbuf[p][pl.ds(j * L, L)]
                for i in range(L):
                    sv = vv[i]
                    e = j * L + i
                    for g in range(D // L):
                        sl = pl.ds(g * L, L)
                        gbuf[p][e, sl] = gbuf[p][e, sl] * sv
            pltpu.async_copy(gbuf[p], acc.at[lidxv[p]], sem_s[p], add=True)

    for t in range(NB):
        stage_a(t, t)
    stage_b(0, 0, first=True)

    def body3(u, carry):
        for q in range(NB):
            t = NB * u + q
            stage_b(t + 1, (q + 1) % NB)
            stage_c(t, q)
            stage_a(t + NB, q)
        return carry

    lax.fori_loop(0, (nt + NB - 1) // NB, body3, 0)

    t_last = nt - 1
    for p in range(NB):
        tp = t_last - jnp.mod(t_last - p, NB)

        @pl.when(tp >= 0)
        def _drain_tail(p=p):
            drain_scatter(p)

    plsc.subcore_barrier()
    pltpu.sync_copy(acc.at[pl.ds(s * R_TILE, R_TILE)],
                    out.at[pl.ds(c * R_CORE + s * R_TILE, R_TILE)])


@jax.jit
def _layer(ego, edat, vals, prm):
    mesh = plsc.VectorSubcoreMesh(core_axis_name="c", subcore_axis_name="s")
    return pl.kernel(
        _layer_body,
        out_type=jax.ShapeDtypeStruct((N_PAD, D), jnp.float32),
        mesh=mesh,
        scratch_types=(
            [pltpu.VMEM((2, CH), jnp.int32)] * NB
            + [pltpu.VMEM((CH,), jnp.float32)] * NB
            + [pltpu.VMEM((CH,), jnp.int32)] * NB
            + [pltpu.VMEM((CH, D), jnp.float32)] * NB
            + [pltpu.VMEM((L, D), jnp.float32),
               pltpu.VMEM((L,), jnp.int32),
               pltpu.VMEM_SHARED((ACC_R, D), jnp.float32)]
            + [pltpu.SemaphoreType.DMA] * (3 * NB)
        ),
    )(ego, edat, vals, prm)


def _mean_body(a, b, c, o):
    o[...] = (a[...] + b[...] + c[...]) * (1.0 / 3.0)


@jax.jit
def _mean3(e1, e2, e3):
    blk = pl.BlockSpec((1024, D), lambda i: (i, 0))
    return pl.pallas_call(
        _mean_body,
        grid=(N_PAD // 1024,),
        in_specs=[blk, blk, blk],
        out_specs=blk,
        out_shape=jax.ShapeDtypeStruct((N_PAD, D), jnp.float32),
    )(e1, e2, e3)


def kernel(user_emb, item_emb, adj_rows, adj_cols, adj_vals):
    ego0 = jnp.concatenate(
        [user_emb, item_emb,
         jnp.zeros((N_PAD - N_TOTAL, D), jnp.float32)], axis=0)
    rows = adj_rows.astype(jnp.int32)
    cols = adj_cols.astype(jnp.int32)
    vals = adj_vals.astype(jnp.float32)
    pad = E_PAD - NNZ
    rows_p = jnp.concatenate([rows, jnp.full((pad,), N_TOTAL - 1, jnp.int32)])
    cols_p = jnp.concatenate([cols, jnp.zeros((pad,), jnp.int32)])
    vals_p = jnp.concatenate([vals, jnp.zeros((pad,), jnp.float32)])
    edat = jnp.stack([cols_p, rows_p], axis=0)
    split = jnp.searchsorted(rows, R_CORE, side="left").astype(jnp.int32)
    prm = jnp.full((L,), split, jnp.int32)

    ego = ego0
    outs = []
    for _ in range(LAYERS):
        ego = _layer(ego, edat, vals_p, prm)
        outs.append(ego)
    mean = _mean3(*outs)
    return (mean[:NUM_USERS], mean[NUM_USERS:N_TOTAL])
